# baseline (speedup 1.0000x reference)
import functools

import jax
import jax.numpy as jnp
from jax import lax
from jax.experimental import pallas as pl
from jax.experimental.pallas import tpu as pltpu
from jax.experimental.pallas import tpu_sc as plsc

_D = 128
_DP = _D // 2
_G = 64
_NW = 32
_FPW = _DP // _NW
_ECHUNK = 8192
_LANES = 16

def _pack_rows(b):
    n = b.shape[1]
    rt = b.astype(jnp.bfloat16).astype(jnp.float32)
    bits = lax.bitcast_convert_type(rt, jnp.uint32).reshape(_DP, 2, n)
    pk = (bits[:, 0, :] >> 16) | ((bits[:, 1, :] >> 16) << 16)
    return lax.bitcast_convert_type(pk, jnp.int32)


def _unpack_rows(pk):
    n = pk.shape[1]
    bits = lax.bitcast_convert_type(pk, jnp.uint32)
    lo = lax.bitcast_convert_type(bits << 16, jnp.float32)
    hi = lax.bitcast_convert_type((bits >> 16) << 16, jnp.float32)
    return jnp.concatenate([lo[:, None, :], hi[:, None, :]], axis=1
                           ).reshape(_D, n)


def _bn_rows(z, g, b):
    mu = jnp.mean(z, axis=1, keepdims=True)
    var = jnp.mean((z - mu) ** 2, axis=1, keepdims=True)
    return g * (z - mu) * lax.rsqrt(var + 1e-5) + b


def _onehot(batch2d, n):
    ids = lax.broadcasted_iota(jnp.int32, (n, _G), 1)
    return (batch2d == ids).astype(jnp.float32)


def _head_body(xT_ref, w1a_ref, b1a_ref, g1a_ref, be1a_ref, w1b_ref, b1b_ref,
               t1_ref, t2_ref, batch_ref, bpk_ref, ct_ref, s_ref):
    n = xT_ref.shape[1]
    z = jnp.dot(w1a_ref[:].T, xT_ref[:], preferred_element_type=jnp.float32)
    z = z + b1a_ref[:]
    zn = jnp.maximum(_bn_rows(z, g1a_ref[:], be1a_ref[:]), 0.0)
    h = jnp.dot(w1b_ref[:].T, zn, preferred_element_type=jnp.float32) + b1b_ref[:]
    oh = _onehot(batch_ref[:], n)
    s_ref[:] = lax.dot_general(oh, h, (((0,), (1,)), ((), ())),
                               preferred_element_type=jnp.float32)
    a = jnp.dot(t1_ref[:].T, h, preferred_element_type=jnp.float32)
    b = jnp.dot(t2_ref[:].T, h, preferred_element_type=jnp.float32)
    bpk_ref[:] = _pack_rows(b)
    ct_ref[:] = a - b


def _mid_body(ct_ref, mpk_ref, tee_ref, g_ref, b_ref, t1_ref, t2_ref,
              batch_ref, h_ref, bpk_ref, ctout_ref, s_ref):
    n = ct_ref.shape[1]
    mt = _unpack_rows(mpk_ref[:])
    agg = jnp.maximum(tee_ref[:] * (ct_ref[:] + mt), 0.0)
    h = jnp.maximum(_bn_rows(agg, g_ref[:], b_ref[:]), 0.0)
    h_ref[:] = h
    oh = _onehot(batch_ref[:], n)
    s_ref[:] = lax.dot_general(oh, h, (((0,), (1,)), ((), ())),
                               preferred_element_type=jnp.float32)
    a = jnp.dot(t1_ref[:].T, h, preferred_element_type=jnp.float32)
    b = jnp.dot(t2_ref[:].T, h, preferred_element_type=jnp.float32)
    bpk_ref[:] = _pack_rows(b)
    ctout_ref[:] = a - b


def _final_body(ct_ref, mpk_ref, tee_ref, g_ref, b_ref, h1_ref, h2_ref,
                batch_ref, s0_ref, s1_ref, s2_ref, wp_ref, bp_ref,
                np_ref, gp_ref):
    n = ct_ref.shape[1]
    mt = _unpack_rows(mpk_ref[:])
    agg = jnp.maximum(tee_ref[:] * (ct_ref[:] + mt), 0.0)
    h3 = jnp.maximum(_bn_rows(agg, g_ref[:], b_ref[:]), 0.0)
    np_ref[:] = h1_ref[:] + h2_ref[:] + h3
    oh = _onehot(batch_ref[:], n)
    s3 = lax.dot_general(oh, h3, (((0,), (1,)), ((), ())),
                         preferred_element_type=jnp.float32)
    ones = jnp.ones((1, n), dtype=jnp.float32)
    cnt = lax.dot_general(oh, ones, (((0,), (1,)), ((), ())),
                          preferred_element_type=jnp.float32)
    inv = 1.0 / jnp.maximum(cnt, 1.0)
    gp = jnp.zeros((_G, _D), dtype=jnp.float32)
    for l, s in enumerate((s0_ref[:], s1_ref[:], s2_ref[:], s3)):
        gp = gp + jnp.dot(s * inv, wp_ref[l],
                          preferred_element_type=jnp.float32) + bp_ref[l]
    gp_ref[:] = gp


def _tc_head(xT, w1a, b1a, g1a, be1a, w1b, b1b, t1, t2, batch2d):
    n = xT.shape[1]
    return pl.pallas_call(
        _head_body,
        out_shape=(
            jax.ShapeDtypeStruct((_DP, n), jnp.int32),
            jax.ShapeDtypeStruct((_D, n), jnp.float32),
            jax.ShapeDtypeStruct((_G, _D), jnp.float32),
        ),
    )(xT, w1a, b1a, g1a, be1a, w1b, b1b, t1, t2, batch2d)


def _tc_mid(ct, mpk, tee, g, b, t1, t2, batch2d):
    n = ct.shape[1]
    return pl.pallas_call(
        _mid_body,
        out_shape=(
            jax.ShapeDtypeStruct((_D, n), jnp.float32),
            jax.ShapeDtypeStruct((_DP, n), jnp.int32),
            jax.ShapeDtypeStruct((_D, n), jnp.float32),
            jax.ShapeDtypeStruct((_G, _D), jnp.float32),
        ),
    )(ct, mpk, tee, g, b, t1, t2, batch2d)


def _tc_final(ct, mpk, tee, g, b, h1, h2, batch2d, s0, s1, s2, wp, bp):
    n = ct.shape[1]
    return pl.pallas_call(
        _final_body,
        out_shape=(
            jax.ShapeDtypeStruct((_D, n), jnp.float32),
            jax.ShapeDtypeStruct((_G, _D), jnp.float32),
        ),
    )(ct, mpk, tee, g, b, h1, h2, batch2d, s0, s1, s2, wp, bp)


def _sc_segmax(bpk, src_p, dst_p):
    n = bpk.shape[1]
    epad = src_p.shape[0]
    nchunks = epad // _ECHUNK
    slab = _FPW * n
    mesh = plsc.VectorSubcoreMesh(core_axis_name="c", subcore_axis_name="s")
    unroll = 8

    @functools.partial(
        pl.kernel,
        out_type=jax.ShapeDtypeStruct((_DP * n,), jnp.int32),
        mesh=mesh,
        scratch_types=(
            [pltpu.VMEM((n,), jnp.int32) for _ in range(_FPW)]
            + [pltpu.VMEM((n,), jnp.int32) for _ in range(_FPW)]
            + [
                pltpu.VMEM((_ECHUNK,), jnp.int32),
                pltpu.VMEM((_ECHUNK,), jnp.int32),
                pltpu.VMEM((n,), jnp.int32),
            ]
        ),
        compiler_params=pltpu.CompilerParams(needs_layout_passes=False),
    )
    def k(bpk_hbm, src_hbm, dst_hbm, out_hbm, *refs):
        bt_v = refs[:_FPW]
        mt_v = refs[_FPW:2 * _FPW]
        src_v, dst_v, dup_v = refs[2 * _FPW:]
        wid = lax.axis_index("s") * 2 + lax.axis_index("c")
        fbase = wid * slab
        for f in range(_FPW):
            pltpu.sync_copy(bpk_hbm.at[pl.ds(fbase + f * n, n)], bt_v[f])
            pltpu.sync_copy(bpk_hbm.at[pl.ds(fbase + f * n, n)], mt_v[f])

        lanes_iota = lax.iota(jnp.int32, _LANES)

        def pmax(a_i32, b_i32):
            a = plsc.bitcast(a_i32, jnp.bfloat16)
            b = plsc.bitcast(b_i32, jnp.bfloat16)
            return plsc.bitcast(jnp.maximum(a, b), jnp.int32)

        def slow_vec(s16, d16):
            for f in range(_FPW):
                vals = plsc.load_gather(bt_v[f], [s16])

                def need(cur):
                    return pmax(vals, cur) != cur

                def wcond(nd):
                    return jnp.any(nd)

                def wbody(nd):
                    cur = plsc.load_gather(mt_v[f], [d16])
                    plsc.store_scatter(mt_v[f], [d16], pmax(vals, cur),
                                       mask=nd)
                    return need(plsc.load_gather(mt_v[f], [d16]))

                lax.while_loop(wcond, wbody,
                               need(plsc.load_gather(mt_v[f], [d16])))

        def chunk_body(cix, carry):
            pltpu.sync_copy(src_hbm.at[pl.ds(cix * _ECHUNK, _ECHUNK)], src_v)
            pltpu.sync_copy(dst_hbm.at[pl.ds(cix * _ECHUNK, _ECHUNK)], dst_v)

            def vec_body(vix, c2):
                mism = jnp.zeros((_LANES,), jnp.bool_)
                for u in range(unroll):
                    off = (vix * unroll + u) * _LANES
                    s16 = src_v[pl.ds(off, _LANES)]
                    d16 = dst_v[pl.ds(off, _LANES)]
                    plsc.store_scatter(dup_v, [d16], lanes_iota)
                    back = plsc.load_gather(dup_v, [d16])
                    mism = mism | (back != lanes_iota)
                    for f in range(_FPW):
                        vals = plsc.load_gather(bt_v[f], [s16])
                        cur = plsc.load_gather(mt_v[f], [d16])
                        plsc.store_scatter(mt_v[f], [d16], pmax(vals, cur))

                @pl.when(jnp.any(mism))
                def _slow():
                    for u in range(unroll):
                        off = (vix * unroll + u) * _LANES
                        slow_vec(src_v[pl.ds(off, _LANES)],
                                 dst_v[pl.ds(off, _LANES)])
                return c2

            return lax.fori_loop(0, _ECHUNK // (_LANES * unroll), vec_body,
                                 carry)

        lax.fori_loop(0, nchunks, chunk_body, 0)
        for f in range(_FPW):
            pltpu.sync_copy(mt_v[f], out_hbm.at[pl.ds(fbase + f * n, n)])

    return k(bpk.reshape(_DP * n), src_p, dst_p).reshape(_DP, n)


def kernel(x, edge_index, batch, W1a, b1a, g1a, be1a, W1b, b1b,
           theta1, theta2, theta_ee, gbn, bbn, Wp, bp):
    n, d = x.shape
    e = edge_index.shape[1]
    epad = ((e + _ECHUNK - 1) // _ECHUNK) * _ECHUNK

    xT = x.T
    src_p = jnp.concatenate(
        [edge_index[0], jnp.zeros((epad - e,), edge_index.dtype)])
    dst_p = jnp.concatenate(
        [edge_index[1], jnp.zeros((epad - e,), edge_index.dtype)])
    batch2d = batch.reshape(n, 1)

    col = lambda v: v.reshape(d, 1)
    bp4 = bp.reshape(4, 1, d)

    bpk0, ct0, s0 = _tc_head(xT, W1a, col(b1a), col(g1a), col(be1a),
                             W1b, col(b1b), theta1[0], theta2[0], batch2d)
    mpk0 = _sc_segmax(bpk0, src_p, dst_p)
    h1, bpk1, ct1, s1 = _tc_mid(ct0, mpk0, col(theta_ee[0]), col(gbn[0]),
                                col(bbn[0]), theta1[1], theta2[1], batch2d)
    mpk1 = _sc_segmax(bpk1, src_p, dst_p)
    h2, bpk2, ct2, s2 = _tc_mid(ct1, mpk1, col(theta_ee[1]), col(gbn[1]),
                                col(bbn[1]), theta1[2], theta2[2], batch2d)
    mpk2 = _sc_segmax(bpk2, src_p, dst_p)
    node_poolT, gpool = _tc_final(ct2, mpk2, col(theta_ee[2]), col(gbn[2]),
                                  col(bbn[2]), h1, h2, batch2d,
                                  s0, s1, s2, Wp, bp4)
    return node_poolT.T, gpool


# speedup vs baseline: 1.0074x; 1.0074x over previous
import functools

import jax
import jax.numpy as jnp
from jax import lax
from jax.experimental import pallas as pl
from jax.experimental.pallas import tpu as pltpu
from jax.experimental.pallas import tpu_sc as plsc

_D = 128
_DP = _D // 2
_G = 64
_NW = 32
_FPW = _DP // _NW
_ECHUNK = 8192
_LANES = 16

def _pack_rows(b):
    n = b.shape[1]
    rt = b.astype(jnp.bfloat16).astype(jnp.float32)
    bits = lax.bitcast_convert_type(rt, jnp.uint32).reshape(_DP, 2, n)
    pk = (bits[:, 0, :] >> 16) | ((bits[:, 1, :] >> 16) << 16)
    return lax.bitcast_convert_type(pk, jnp.int32)


def _unpack_rows(pk):
    n = pk.shape[1]
    bits = lax.bitcast_convert_type(pk, jnp.uint32)
    lo = lax.bitcast_convert_type(bits << 16, jnp.float32)
    hi = lax.bitcast_convert_type((bits >> 16) << 16, jnp.float32)
    return jnp.concatenate([lo[:, None, :], hi[:, None, :]], axis=1
                           ).reshape(_D, n)


def _bn_rows(z, g, b):
    mu = jnp.mean(z, axis=1, keepdims=True)
    var = jnp.mean((z - mu) ** 2, axis=1, keepdims=True)
    return g * (z - mu) * lax.rsqrt(var + 1e-5) + b


def _onehot(batch2d, n):
    ids = lax.broadcasted_iota(jnp.int32, (n, _G), 1)
    return (batch2d == ids).astype(jnp.float32)


def _head_body(xT_ref, w1a_ref, b1a_ref, g1a_ref, be1a_ref, w1b_ref, b1b_ref,
               t1_ref, t2_ref, batch_ref, bpk_ref, ct_ref, s_ref):
    n = xT_ref.shape[1]
    z = jnp.dot(w1a_ref[:].T, xT_ref[:], preferred_element_type=jnp.float32)
    z = z + b1a_ref[:]
    zn = jnp.maximum(_bn_rows(z, g1a_ref[:], be1a_ref[:]), 0.0)
    h = jnp.dot(w1b_ref[:].T, zn, preferred_element_type=jnp.float32) + b1b_ref[:]
    oh = _onehot(batch_ref[:], n)
    s_ref[:] = lax.dot_general(oh, h, (((0,), (1,)), ((), ())),
                               preferred_element_type=jnp.float32)
    a = jnp.dot(t1_ref[:].T, h, preferred_element_type=jnp.float32)
    b = jnp.dot(t2_ref[:].T, h, preferred_element_type=jnp.float32)
    bpk_ref[:] = _pack_rows(b)
    ct_ref[:] = a - b


def _mid_body(ct_ref, mpk_ref, tee_ref, g_ref, b_ref, t1_ref, t2_ref,
              batch_ref, h_ref, bpk_ref, ctout_ref, s_ref):
    n = ct_ref.shape[1]
    mt = _unpack_rows(mpk_ref[:])
    agg = jnp.maximum(tee_ref[:] * (ct_ref[:] + mt), 0.0)
    h = jnp.maximum(_bn_rows(agg, g_ref[:], b_ref[:]), 0.0)
    h_ref[:] = h
    oh = _onehot(batch_ref[:], n)
    s_ref[:] = lax.dot_general(oh, h, (((0,), (1,)), ((), ())),
                               preferred_element_type=jnp.float32)
    a = jnp.dot(t1_ref[:].T, h, preferred_element_type=jnp.float32)
    b = jnp.dot(t2_ref[:].T, h, preferred_element_type=jnp.float32)
    bpk_ref[:] = _pack_rows(b)
    ctout_ref[:] = a - b


def _final_body(ct_ref, mpk_ref, tee_ref, g_ref, b_ref, h1_ref, h2_ref,
                batch_ref, s0_ref, s1_ref, s2_ref, wp_ref, bp_ref,
                np_ref, gp_ref):
    n = ct_ref.shape[1]
    mt = _unpack_rows(mpk_ref[:])
    agg = jnp.maximum(tee_ref[:] * (ct_ref[:] + mt), 0.0)
    h3 = jnp.maximum(_bn_rows(agg, g_ref[:], b_ref[:]), 0.0)
    np_ref[:] = h1_ref[:] + h2_ref[:] + h3
    oh = _onehot(batch_ref[:], n)
    s3 = lax.dot_general(oh, h3, (((0,), (1,)), ((), ())),
                         preferred_element_type=jnp.float32)
    ones = jnp.ones((1, n), dtype=jnp.float32)
    cnt = lax.dot_general(oh, ones, (((0,), (1,)), ((), ())),
                          preferred_element_type=jnp.float32)
    inv = 1.0 / jnp.maximum(cnt, 1.0)
    gp = jnp.zeros((_G, _D), dtype=jnp.float32)
    for l, s in enumerate((s0_ref[:], s1_ref[:], s2_ref[:], s3)):
        gp = gp + jnp.dot(s * inv, wp_ref[l],
                          preferred_element_type=jnp.float32) + bp_ref[l]
    gp_ref[:] = gp


def _tc_head(xT, w1a, b1a, g1a, be1a, w1b, b1b, t1, t2, batch2d):
    n = xT.shape[1]
    return pl.pallas_call(
        _head_body,
        out_shape=(
            jax.ShapeDtypeStruct((_DP, n), jnp.int32),
            jax.ShapeDtypeStruct((_D, n), jnp.float32),
            jax.ShapeDtypeStruct((_G, _D), jnp.float32),
        ),
    )(xT, w1a, b1a, g1a, be1a, w1b, b1b, t1, t2, batch2d)


def _tc_mid(ct, mpk, tee, g, b, t1, t2, batch2d):
    n = ct.shape[1]
    return pl.pallas_call(
        _mid_body,
        out_shape=(
            jax.ShapeDtypeStruct((_D, n), jnp.float32),
            jax.ShapeDtypeStruct((_DP, n), jnp.int32),
            jax.ShapeDtypeStruct((_D, n), jnp.float32),
            jax.ShapeDtypeStruct((_G, _D), jnp.float32),
        ),
    )(ct, mpk, tee, g, b, t1, t2, batch2d)


def _tc_final(ct, mpk, tee, g, b, h1, h2, batch2d, s0, s1, s2, wp, bp):
    n = ct.shape[1]
    return pl.pallas_call(
        _final_body,
        out_shape=(
            jax.ShapeDtypeStruct((_D, n), jnp.float32),
            jax.ShapeDtypeStruct((_G, _D), jnp.float32),
        ),
    )(ct, mpk, tee, g, b, h1, h2, batch2d, s0, s1, s2, wp, bp)


def _sc_segmax(bpk, src_p, dst_p):
    n = bpk.shape[1]
    epad = src_p.shape[0]
    nchunks = epad // _ECHUNK
    slab = _FPW * n
    mesh = plsc.VectorSubcoreMesh(core_axis_name="c", subcore_axis_name="s")
    unroll = 4

    @functools.partial(
        pl.kernel,
        out_type=jax.ShapeDtypeStruct((_DP * n,), jnp.int32),
        mesh=mesh,
        scratch_types=(
            [pltpu.VMEM((n,), jnp.int32) for _ in range(_FPW)]
            + [pltpu.VMEM((n,), jnp.int32) for _ in range(_FPW)]
            + [
                pltpu.VMEM((_ECHUNK,), jnp.int32),
                pltpu.VMEM((_ECHUNK,), jnp.int32),
                pltpu.VMEM((n,), jnp.int32),
            ]
        ),
        compiler_params=pltpu.CompilerParams(needs_layout_passes=False),
    )
    def k(bpk_hbm, src_hbm, dst_hbm, out_hbm, *refs):
        bt_v = refs[:_FPW]
        mt_v = refs[_FPW:2 * _FPW]
        src_v, dst_v, dup_v = refs[2 * _FPW:]
        wid = lax.axis_index("s") * 2 + lax.axis_index("c")
        fbase = wid * slab
        for f in range(_FPW):
            pltpu.sync_copy(bpk_hbm.at[pl.ds(fbase + f * n, n)], bt_v[f])
            pltpu.sync_copy(bpk_hbm.at[pl.ds(fbase + f * n, n)], mt_v[f])

        lanes_iota = lax.iota(jnp.int32, _LANES)

        def pmax(a_i32, b_i32):
            a = plsc.bitcast(a_i32, jnp.bfloat16)
            b = plsc.bitcast(b_i32, jnp.bfloat16)
            return plsc.bitcast(jnp.maximum(a, b), jnp.int32)

        def slow_vec(s16, d16):
            for f in range(_FPW):
                vals = plsc.load_gather(bt_v[f], [s16])

                def need(cur):
                    return pmax(vals, cur) != cur

                def wcond(nd):
                    return jnp.any(nd)

                def wbody(nd):
                    cur = plsc.load_gather(mt_v[f], [d16])
                    plsc.store_scatter(mt_v[f], [d16], pmax(vals, cur),
                                       mask=nd)
                    return need(plsc.load_gather(mt_v[f], [d16]))

                lax.while_loop(wcond, wbody,
                               need(plsc.load_gather(mt_v[f], [d16])))

        def chunk_body(cix, carry):
            pltpu.sync_copy(src_hbm.at[pl.ds(cix * _ECHUNK, _ECHUNK)], src_v)
            pltpu.sync_copy(dst_hbm.at[pl.ds(cix * _ECHUNK, _ECHUNK)], dst_v)

            def vec_body(vix, c2):
                mism = jnp.zeros((_LANES,), jnp.bool_)
                for u in range(unroll):
                    off = (vix * unroll + u) * _LANES
                    s16 = src_v[pl.ds(off, _LANES)]
                    d16 = dst_v[pl.ds(off, _LANES)]
                    plsc.store_scatter(dup_v, [d16], lanes_iota)
                    back = plsc.load_gather(dup_v, [d16])
                    mism = mism | (back != lanes_iota)
                    for f in range(_FPW):
                        vals = plsc.load_gather(bt_v[f], [s16])
                        cur = plsc.load_gather(mt_v[f], [d16])
                        plsc.store_scatter(mt_v[f], [d16], pmax(vals, cur))

                @pl.when(jnp.any(mism))
                def _slow():
                    for u in range(unroll):
                        off = (vix * unroll + u) * _LANES
                        slow_vec(src_v[pl.ds(off, _LANES)],
                                 dst_v[pl.ds(off, _LANES)])
                return c2

            return lax.fori_loop(0, _ECHUNK // (_LANES * unroll), vec_body,
                                 carry)

        lax.fori_loop(0, nchunks, chunk_body, 0)
        for f in range(_FPW):
            pltpu.sync_copy(mt_v[f], out_hbm.at[pl.ds(fbase + f * n, n)])

    return k(bpk.reshape(_DP * n), src_p, dst_p).reshape(_DP, n)


def kernel(x, edge_index, batch, W1a, b1a, g1a, be1a, W1b, b1b,
           theta1, theta2, theta_ee, gbn, bbn, Wp, bp):
    n, d = x.shape
    e = edge_index.shape[1]
    epad = ((e + _ECHUNK - 1) // _ECHUNK) * _ECHUNK

    xT = x.T
    src_p = jnp.concatenate(
        [edge_index[0], jnp.zeros((epad - e,), edge_index.dtype)])
    dst_p = jnp.concatenate(
        [edge_index[1], jnp.zeros((epad - e,), edge_index.dtype)])
    batch2d = batch.reshape(n, 1)

    col = lambda v: v.reshape(d, 1)
    bp4 = bp.reshape(4, 1, d)

    bpk0, ct0, s0 = _tc_head(xT, W1a, col(b1a), col(g1a), col(be1a),
                             W1b, col(b1b), theta1[0], theta2[0], batch2d)
    mpk0 = _sc_segmax(bpk0, src_p, dst_p)
    h1, bpk1, ct1, s1 = _tc_mid(ct0, mpk0, col(theta_ee[0]), col(gbn[0]),
                                col(bbn[0]), theta1[1], theta2[1], batch2d)
    mpk1 = _sc_segmax(bpk1, src_p, dst_p)
    h2, bpk2, ct2, s2 = _tc_mid(ct1, mpk1, col(theta_ee[1]), col(gbn[1]),
                                col(bbn[1]), theta1[2], theta2[2], batch2d)
    mpk2 = _sc_segmax(bpk2, src_p, dst_p)
    node_poolT, gpool = _tc_final(ct2, mpk2, col(theta_ee[2]), col(gbn[2]),
                                  col(bbn[2]), h1, h2, batch2d,
                                  s0, s1, s2, Wp, bp4)
    return node_poolT.T, gpool


# speedup vs baseline: 1.0441x; 1.0364x over previous
import functools

import jax
import jax.numpy as jnp
from jax import lax
from jax.experimental import pallas as pl
from jax.experimental.pallas import tpu as pltpu
from jax.experimental.pallas import tpu_sc as plsc

_D = 128
_DP = _D // 2
_G = 64
_NW = 32
_FPW = _DP // _NW
_ECHUNK = 16384
_LANES = 16

def _pack_rows(b):
    n = b.shape[1]
    rt = b.astype(jnp.bfloat16).astype(jnp.float32)
    bits = lax.bitcast_convert_type(rt, jnp.uint32).reshape(_DP, 2, n)
    pk = (bits[:, 0, :] >> 16) | ((bits[:, 1, :] >> 16) << 16)
    return lax.bitcast_convert_type(pk, jnp.int32)


def _unpack_rows(pk):
    n = pk.shape[1]
    bits = lax.bitcast_convert_type(pk, jnp.uint32)
    lo = lax.bitcast_convert_type(bits << 16, jnp.float32)
    hi = lax.bitcast_convert_type((bits >> 16) << 16, jnp.float32)
    return jnp.concatenate([lo[:, None, :], hi[:, None, :]], axis=1
                           ).reshape(_D, n)


def _bn_rows(z, g, b):
    mu = jnp.mean(z, axis=1, keepdims=True)
    var = jnp.mean((z - mu) ** 2, axis=1, keepdims=True)
    return g * (z - mu) * lax.rsqrt(var + 1e-5) + b


def _onehot(batch2d, n):
    ids = lax.broadcasted_iota(jnp.int32, (n, _G), 1)
    return (batch2d == ids).astype(jnp.float32)


def _head_body(xT_ref, w1a_ref, b1a_ref, g1a_ref, be1a_ref, w1b_ref, b1b_ref,
               t1_ref, t2_ref, batch_ref, bpk_ref, ct_ref, s_ref):
    n = xT_ref.shape[1]
    z = jnp.dot(w1a_ref[:].T, xT_ref[:], preferred_element_type=jnp.float32)
    z = z + b1a_ref[:]
    zn = jnp.maximum(_bn_rows(z, g1a_ref[:], be1a_ref[:]), 0.0)
    h = jnp.dot(w1b_ref[:].T, zn, preferred_element_type=jnp.float32) + b1b_ref[:]
    oh = _onehot(batch_ref[:], n)
    s_ref[:] = lax.dot_general(oh, h, (((0,), (1,)), ((), ())),
                               preferred_element_type=jnp.float32)
    a = jnp.dot(t1_ref[:].T, h, preferred_element_type=jnp.float32)
    b = jnp.dot(t2_ref[:].T, h, preferred_element_type=jnp.float32)
    bpk_ref[:] = _pack_rows(b)
    ct_ref[:] = a - b


def _mid_body(ct_ref, mpk_ref, tee_ref, g_ref, b_ref, t1_ref, t2_ref,
              batch_ref, h_ref, bpk_ref, ctout_ref, s_ref):
    n = ct_ref.shape[1]
    mt = _unpack_rows(mpk_ref[:])
    agg = jnp.maximum(tee_ref[:] * (ct_ref[:] + mt), 0.0)
    h = jnp.maximum(_bn_rows(agg, g_ref[:], b_ref[:]), 0.0)
    h_ref[:] = h
    oh = _onehot(batch_ref[:], n)
    s_ref[:] = lax.dot_general(oh, h, (((0,), (1,)), ((), ())),
                               preferred_element_type=jnp.float32)
    a = jnp.dot(t1_ref[:].T, h, preferred_element_type=jnp.float32)
    b = jnp.dot(t2_ref[:].T, h, preferred_element_type=jnp.float32)
    bpk_ref[:] = _pack_rows(b)
    ctout_ref[:] = a - b


def _final_body(ct_ref, mpk_ref, tee_ref, g_ref, b_ref, h1_ref, h2_ref,
                batch_ref, s0_ref, s1_ref, s2_ref, wp_ref, bp_ref,
                np_ref, gp_ref):
    n = ct_ref.shape[1]
    mt = _unpack_rows(mpk_ref[:])
    agg = jnp.maximum(tee_ref[:] * (ct_ref[:] + mt), 0.0)
    h3 = jnp.maximum(_bn_rows(agg, g_ref[:], b_ref[:]), 0.0)
    np_ref[:] = h1_ref[:] + h2_ref[:] + h3
    oh = _onehot(batch_ref[:], n)
    s3 = lax.dot_general(oh, h3, (((0,), (1,)), ((), ())),
                         preferred_element_type=jnp.float32)
    ones = jnp.ones((1, n), dtype=jnp.float32)
    cnt = lax.dot_general(oh, ones, (((0,), (1,)), ((), ())),
                          preferred_element_type=jnp.float32)
    inv = 1.0 / jnp.maximum(cnt, 1.0)
    gp = jnp.zeros((_G, _D), dtype=jnp.float32)
    for l, s in enumerate((s0_ref[:], s1_ref[:], s2_ref[:], s3)):
        gp = gp + jnp.dot(s * inv, wp_ref[l],
                          preferred_element_type=jnp.float32) + bp_ref[l]
    gp_ref[:] = gp


def _tc_head(xT, w1a, b1a, g1a, be1a, w1b, b1b, t1, t2, batch2d):
    n = xT.shape[1]
    return pl.pallas_call(
        _head_body,
        out_shape=(
            jax.ShapeDtypeStruct((_DP, n), jnp.int32),
            jax.ShapeDtypeStruct((_D, n), jnp.float32),
            jax.ShapeDtypeStruct((_G, _D), jnp.float32),
        ),
    )(xT, w1a, b1a, g1a, be1a, w1b, b1b, t1, t2, batch2d)


def _tc_mid(ct, mpk, tee, g, b, t1, t2, batch2d):
    n = ct.shape[1]
    return pl.pallas_call(
        _mid_body,
        out_shape=(
            jax.ShapeDtypeStruct((_D, n), jnp.float32),
            jax.ShapeDtypeStruct((_DP, n), jnp.int32),
            jax.ShapeDtypeStruct((_D, n), jnp.float32),
            jax.ShapeDtypeStruct((_G, _D), jnp.float32),
        ),
    )(ct, mpk, tee, g, b, t1, t2, batch2d)


def _tc_final(ct, mpk, tee, g, b, h1, h2, batch2d, s0, s1, s2, wp, bp):
    n = ct.shape[1]
    return pl.pallas_call(
        _final_body,
        out_shape=(
            jax.ShapeDtypeStruct((_D, n), jnp.float32),
            jax.ShapeDtypeStruct((_G, _D), jnp.float32),
        ),
    )(ct, mpk, tee, g, b, h1, h2, batch2d, s0, s1, s2, wp, bp)


def _sc_segmax(bpk, src_p, dst_p):
    n = bpk.shape[1]
    epad = src_p.shape[0]
    nchunks = epad // _ECHUNK
    slab = _FPW * n
    mesh = plsc.VectorSubcoreMesh(core_axis_name="c", subcore_axis_name="s")
    unroll = 4

    @functools.partial(
        pl.kernel,
        out_type=jax.ShapeDtypeStruct((_DP * n,), jnp.int32),
        mesh=mesh,
        scratch_types=(
            [pltpu.VMEM((n,), jnp.int32) for _ in range(_FPW)]
            + [pltpu.VMEM((n,), jnp.int32) for _ in range(_FPW)]
            + [
                pltpu.VMEM((_ECHUNK,), jnp.int32),
                pltpu.VMEM((_ECHUNK,), jnp.int32),
                pltpu.VMEM((n,), jnp.int32),
            ]
        ),
        compiler_params=pltpu.CompilerParams(needs_layout_passes=False),
    )
    def k(bpk_hbm, src_hbm, dst_hbm, out_hbm, *refs):
        bt_v = refs[:_FPW]
        mt_v = refs[_FPW:2 * _FPW]
        src_v, dst_v, dup_v = refs[2 * _FPW:]
        wid = lax.axis_index("s") * 2 + lax.axis_index("c")
        fbase = wid * slab
        for f in range(_FPW):
            pltpu.sync_copy(bpk_hbm.at[pl.ds(fbase + f * n, n)], bt_v[f])
            pltpu.sync_copy(bpk_hbm.at[pl.ds(fbase + f * n, n)], mt_v[f])

        lanes_iota = lax.iota(jnp.int32, _LANES)

        def pmax(a_i32, b_i32):
            a = plsc.bitcast(a_i32, jnp.bfloat16)
            b = plsc.bitcast(b_i32, jnp.bfloat16)
            return plsc.bitcast(jnp.maximum(a, b), jnp.int32)

        def slow_vec(s16, d16):
            for f in range(_FPW):
                vals = plsc.load_gather(bt_v[f], [s16])

                def need(cur):
                    return pmax(vals, cur) != cur

                def wcond(nd):
                    return jnp.any(nd)

                def wbody(nd):
                    cur = plsc.load_gather(mt_v[f], [d16])
                    plsc.store_scatter(mt_v[f], [d16], pmax(vals, cur),
                                       mask=nd)
                    return need(plsc.load_gather(mt_v[f], [d16]))

                lax.while_loop(wcond, wbody,
                               need(plsc.load_gather(mt_v[f], [d16])))

        def chunk_body(cix, carry):
            pltpu.sync_copy(src_hbm.at[pl.ds(cix * _ECHUNK, _ECHUNK)], src_v)
            pltpu.sync_copy(dst_hbm.at[pl.ds(cix * _ECHUNK, _ECHUNK)], dst_v)

            def vec_body(vix, c2):
                mism = jnp.zeros((_LANES,), jnp.bool_)
                for u in range(unroll):
                    off = (vix * unroll + u) * _LANES
                    s16 = src_v[pl.ds(off, _LANES)]
                    d16 = dst_v[pl.ds(off, _LANES)]
                    plsc.store_scatter(dup_v, [d16], lanes_iota)
                    back = plsc.load_gather(dup_v, [d16])
                    mism = mism | (back != lanes_iota)
                    for f in range(_FPW):
                        vals = plsc.load_gather(bt_v[f], [s16])
                        cur = plsc.load_gather(mt_v[f], [d16])
                        plsc.store_scatter(mt_v[f], [d16], pmax(vals, cur))

                @pl.when(jnp.any(mism))
                def _slow():
                    for u in range(unroll):
                        off = (vix * unroll + u) * _LANES
                        slow_vec(src_v[pl.ds(off, _LANES)],
                                 dst_v[pl.ds(off, _LANES)])
                return c2

            return lax.fori_loop(0, _ECHUNK // (_LANES * unroll), vec_body,
                                 carry)

        lax.fori_loop(0, nchunks, chunk_body, 0)
        for f in range(_FPW):
            pltpu.sync_copy(mt_v[f], out_hbm.at[pl.ds(fbase + f * n, n)])

    return k(bpk.reshape(_DP * n), src_p, dst_p).reshape(_DP, n)


def kernel(x, edge_index, batch, W1a, b1a, g1a, be1a, W1b, b1b,
           theta1, theta2, theta_ee, gbn, bbn, Wp, bp):
    n, d = x.shape
    e = edge_index.shape[1]
    epad = ((e + _ECHUNK - 1) // _ECHUNK) * _ECHUNK

    xT = x.T
    src_p = jnp.concatenate(
        [edge_index[0], jnp.zeros((epad - e,), edge_index.dtype)])
    dst_p = jnp.concatenate(
        [edge_index[1], jnp.zeros((epad - e,), edge_index.dtype)])
    batch2d = batch.reshape(n, 1)

    col = lambda v: v.reshape(d, 1)
    bp4 = bp.reshape(4, 1, d)

    bpk0, ct0, s0 = _tc_head(xT, W1a, col(b1a), col(g1a), col(be1a),
                             W1b, col(b1b), theta1[0], theta2[0], batch2d)
    mpk0 = _sc_segmax(bpk0, src_p, dst_p)
    h1, bpk1, ct1, s1 = _tc_mid(ct0, mpk0, col(theta_ee[0]), col(gbn[0]),
                                col(bbn[0]), theta1[1], theta2[1], batch2d)
    mpk1 = _sc_segmax(bpk1, src_p, dst_p)
    h2, bpk2, ct2, s2 = _tc_mid(ct1, mpk1, col(theta_ee[1]), col(gbn[1]),
                                col(bbn[1]), theta1[2], theta2[2], batch2d)
    mpk2 = _sc_segmax(bpk2, src_p, dst_p)
    node_poolT, gpool = _tc_final(ct2, mpk2, col(theta_ee[2]), col(gbn[2]),
                                  col(bbn[2]), h1, h2, batch2d,
                                  s0, s1, s2, Wp, bp4)
    return node_poolT.T, gpool


# speedup vs baseline: 1.0639x; 1.0189x over previous
import functools

import jax
import jax.numpy as jnp
from jax import lax
from jax.experimental import pallas as pl
from jax.experimental.pallas import tpu as pltpu
from jax.experimental.pallas import tpu_sc as plsc

_D = 128
_DP = _D // 2
_G = 64
_NW = 32
_FPW = _DP // _NW
_ECHUNK = 32768
_LANES = 16

def _pack_rows(b):
    n = b.shape[1]
    rt = b.astype(jnp.bfloat16).astype(jnp.float32)
    bits = lax.bitcast_convert_type(rt, jnp.uint32).reshape(_DP, 2, n)
    pk = (bits[:, 0, :] >> 16) | ((bits[:, 1, :] >> 16) << 16)
    return lax.bitcast_convert_type(pk, jnp.int32)


def _unpack_rows(pk):
    n = pk.shape[1]
    bits = lax.bitcast_convert_type(pk, jnp.uint32)
    lo = lax.bitcast_convert_type(bits << 16, jnp.float32)
    hi = lax.bitcast_convert_type((bits >> 16) << 16, jnp.float32)
    return jnp.concatenate([lo[:, None, :], hi[:, None, :]], axis=1
                           ).reshape(_D, n)


def _bn_rows(z, g, b):
    mu = jnp.mean(z, axis=1, keepdims=True)
    var = jnp.mean((z - mu) ** 2, axis=1, keepdims=True)
    return g * (z - mu) * lax.rsqrt(var + 1e-5) + b


def _onehot(batch2d, n):
    ids = lax.broadcasted_iota(jnp.int32, (n, _G), 1)
    return (batch2d == ids).astype(jnp.float32)


def _head_body(xT_ref, w1a_ref, b1a_ref, g1a_ref, be1a_ref, w1b_ref, b1b_ref,
               t1_ref, t2_ref, batch_ref, bpk_ref, ct_ref, s_ref):
    n = xT_ref.shape[1]
    z = jnp.dot(w1a_ref[:].T, xT_ref[:], preferred_element_type=jnp.float32)
    z = z + b1a_ref[:]
    zn = jnp.maximum(_bn_rows(z, g1a_ref[:], be1a_ref[:]), 0.0)
    h = jnp.dot(w1b_ref[:].T, zn, preferred_element_type=jnp.float32) + b1b_ref[:]
    oh = _onehot(batch_ref[:], n)
    s_ref[:] = lax.dot_general(oh, h, (((0,), (1,)), ((), ())),
                               preferred_element_type=jnp.float32)
    a = jnp.dot(t1_ref[:].T, h, preferred_element_type=jnp.float32)
    b = jnp.dot(t2_ref[:].T, h, preferred_element_type=jnp.float32)
    bpk_ref[:] = _pack_rows(b)
    ct_ref[:] = a - b


def _mid_body(ct_ref, mpk_ref, tee_ref, g_ref, b_ref, t1_ref, t2_ref,
              batch_ref, h_ref, bpk_ref, ctout_ref, s_ref):
    n = ct_ref.shape[1]
    mt = _unpack_rows(mpk_ref[:])
    agg = jnp.maximum(tee_ref[:] * (ct_ref[:] + mt), 0.0)
    h = jnp.maximum(_bn_rows(agg, g_ref[:], b_ref[:]), 0.0)
    h_ref[:] = h
    oh = _onehot(batch_ref[:], n)
    s_ref[:] = lax.dot_general(oh, h, (((0,), (1,)), ((), ())),
                               preferred_element_type=jnp.float32)
    a = jnp.dot(t1_ref[:].T, h, preferred_element_type=jnp.float32)
    b = jnp.dot(t2_ref[:].T, h, preferred_element_type=jnp.float32)
    bpk_ref[:] = _pack_rows(b)
    ctout_ref[:] = a - b


def _final_body(ct_ref, mpk_ref, tee_ref, g_ref, b_ref, h1_ref, h2_ref,
                batch_ref, s0_ref, s1_ref, s2_ref, wp_ref, bp_ref,
                np_ref, gp_ref):
    n = ct_ref.shape[1]
    mt = _unpack_rows(mpk_ref[:])
    agg = jnp.maximum(tee_ref[:] * (ct_ref[:] + mt), 0.0)
    h3 = jnp.maximum(_bn_rows(agg, g_ref[:], b_ref[:]), 0.0)
    np_ref[:] = h1_ref[:] + h2_ref[:] + h3
    oh = _onehot(batch_ref[:], n)
    s3 = lax.dot_general(oh, h3, (((0,), (1,)), ((), ())),
                         preferred_element_type=jnp.float32)
    ones = jnp.ones((1, n), dtype=jnp.float32)
    cnt = lax.dot_general(oh, ones, (((0,), (1,)), ((), ())),
                          preferred_element_type=jnp.float32)
    inv = 1.0 / jnp.maximum(cnt, 1.0)
    gp = jnp.zeros((_G, _D), dtype=jnp.float32)
    for l, s in enumerate((s0_ref[:], s1_ref[:], s2_ref[:], s3)):
        gp = gp + jnp.dot(s * inv, wp_ref[l],
                          preferred_element_type=jnp.float32) + bp_ref[l]
    gp_ref[:] = gp


def _tc_head(xT, w1a, b1a, g1a, be1a, w1b, b1b, t1, t2, batch2d):
    n = xT.shape[1]
    return pl.pallas_call(
        _head_body,
        out_shape=(
            jax.ShapeDtypeStruct((_DP, n), jnp.int32),
            jax.ShapeDtypeStruct((_D, n), jnp.float32),
            jax.ShapeDtypeStruct((_G, _D), jnp.float32),
        ),
    )(xT, w1a, b1a, g1a, be1a, w1b, b1b, t1, t2, batch2d)


def _tc_mid(ct, mpk, tee, g, b, t1, t2, batch2d):
    n = ct.shape[1]
    return pl.pallas_call(
        _mid_body,
        out_shape=(
            jax.ShapeDtypeStruct((_D, n), jnp.float32),
            jax.ShapeDtypeStruct((_DP, n), jnp.int32),
            jax.ShapeDtypeStruct((_D, n), jnp.float32),
            jax.ShapeDtypeStruct((_G, _D), jnp.float32),
        ),
    )(ct, mpk, tee, g, b, t1, t2, batch2d)


def _tc_final(ct, mpk, tee, g, b, h1, h2, batch2d, s0, s1, s2, wp, bp):
    n = ct.shape[1]
    return pl.pallas_call(
        _final_body,
        out_shape=(
            jax.ShapeDtypeStruct((_D, n), jnp.float32),
            jax.ShapeDtypeStruct((_G, _D), jnp.float32),
        ),
    )(ct, mpk, tee, g, b, h1, h2, batch2d, s0, s1, s2, wp, bp)


def _sc_segmax(bpk, src_p, dst_p):
    n = bpk.shape[1]
    epad = src_p.shape[0]
    nchunks = epad // _ECHUNK
    slab = _FPW * n
    mesh = plsc.VectorSubcoreMesh(core_axis_name="c", subcore_axis_name="s")
    unroll = 4

    @functools.partial(
        pl.kernel,
        out_type=jax.ShapeDtypeStruct((_DP * n,), jnp.int32),
        mesh=mesh,
        scratch_types=(
            [pltpu.VMEM((n,), jnp.int32) for _ in range(_FPW)]
            + [pltpu.VMEM((n,), jnp.int32) for _ in range(_FPW)]
            + [
                pltpu.VMEM((_ECHUNK,), jnp.int32),
                pltpu.VMEM((_ECHUNK,), jnp.int32),
                pltpu.VMEM((n,), jnp.int32),
            ]
        ),
        compiler_params=pltpu.CompilerParams(needs_layout_passes=False),
    )
    def k(bpk_hbm, src_hbm, dst_hbm, out_hbm, *refs):
        bt_v = refs[:_FPW]
        mt_v = refs[_FPW:2 * _FPW]
        src_v, dst_v, dup_v = refs[2 * _FPW:]
        wid = lax.axis_index("s") * 2 + lax.axis_index("c")
        fbase = wid * slab
        for f in range(_FPW):
            pltpu.sync_copy(bpk_hbm.at[pl.ds(fbase + f * n, n)], bt_v[f])
            pltpu.sync_copy(bpk_hbm.at[pl.ds(fbase + f * n, n)], mt_v[f])

        lanes_iota = lax.iota(jnp.int32, _LANES)

        def pmax(a_i32, b_i32):
            a = plsc.bitcast(a_i32, jnp.bfloat16)
            b = plsc.bitcast(b_i32, jnp.bfloat16)
            return plsc.bitcast(jnp.maximum(a, b), jnp.int32)

        def slow_vec(s16, d16):
            for f in range(_FPW):
                vals = plsc.load_gather(bt_v[f], [s16])

                def need(cur):
                    return pmax(vals, cur) != cur

                def wcond(nd):
                    return jnp.any(nd)

                def wbody(nd):
                    cur = plsc.load_gather(mt_v[f], [d16])
                    plsc.store_scatter(mt_v[f], [d16], pmax(vals, cur),
                                       mask=nd)
                    return need(plsc.load_gather(mt_v[f], [d16]))

                lax.while_loop(wcond, wbody,
                               need(plsc.load_gather(mt_v[f], [d16])))

        def chunk_body(cix, carry):
            pltpu.sync_copy(src_hbm.at[pl.ds(cix * _ECHUNK, _ECHUNK)], src_v)
            pltpu.sync_copy(dst_hbm.at[pl.ds(cix * _ECHUNK, _ECHUNK)], dst_v)

            def vec_body(vix, c2):
                mism = jnp.zeros((_LANES,), jnp.bool_)
                for u in range(unroll):
                    off = (vix * unroll + u) * _LANES
                    s16 = src_v[pl.ds(off, _LANES)]
                    d16 = dst_v[pl.ds(off, _LANES)]
                    plsc.store_scatter(dup_v, [d16], lanes_iota)
                    back = plsc.load_gather(dup_v, [d16])
                    mism = mism | (back != lanes_iota)
                    for f in range(_FPW):
                        vals = plsc.load_gather(bt_v[f], [s16])
                        cur = plsc.load_gather(mt_v[f], [d16])
                        plsc.store_scatter(mt_v[f], [d16], pmax(vals, cur))

                @pl.when(jnp.any(mism))
                def _slow():
                    for u in range(unroll):
                        off = (vix * unroll + u) * _LANES
                        slow_vec(src_v[pl.ds(off, _LANES)],
                                 dst_v[pl.ds(off, _LANES)])
                return c2

            return lax.fori_loop(0, _ECHUNK // (_LANES * unroll), vec_body,
                                 carry)

        lax.fori_loop(0, nchunks, chunk_body, 0)
        for f in range(_FPW):
            pltpu.sync_copy(mt_v[f], out_hbm.at[pl.ds(fbase + f * n, n)])

    return k(bpk.reshape(_DP * n), src_p, dst_p).reshape(_DP, n)


def kernel(x, edge_index, batch, W1a, b1a, g1a, be1a, W1b, b1b,
           theta1, theta2, theta_ee, gbn, bbn, Wp, bp):
    n, d = x.shape
    e = edge_index.shape[1]
    epad = ((e + _ECHUNK - 1) // _ECHUNK) * _ECHUNK

    xT = x.T
    src_p = jnp.concatenate(
        [edge_index[0], jnp.zeros((epad - e,), edge_index.dtype)])
    dst_p = jnp.concatenate(
        [edge_index[1], jnp.zeros((epad - e,), edge_index.dtype)])
    batch2d = batch.reshape(n, 1)

    col = lambda v: v.reshape(d, 1)
    bp4 = bp.reshape(4, 1, d)

    bpk0, ct0, s0 = _tc_head(xT, W1a, col(b1a), col(g1a), col(be1a),
                             W1b, col(b1b), theta1[0], theta2[0], batch2d)
    mpk0 = _sc_segmax(bpk0, src_p, dst_p)
    h1, bpk1, ct1, s1 = _tc_mid(ct0, mpk0, col(theta_ee[0]), col(gbn[0]),
                                col(bbn[0]), theta1[1], theta2[1], batch2d)
    mpk1 = _sc_segmax(bpk1, src_p, dst_p)
    h2, bpk2, ct2, s2 = _tc_mid(ct1, mpk1, col(theta_ee[1]), col(gbn[1]),
                                col(bbn[1]), theta1[2], theta2[2], batch2d)
    mpk2 = _sc_segmax(bpk2, src_p, dst_p)
    node_poolT, gpool = _tc_final(ct2, mpk2, col(theta_ee[2]), col(gbn[2]),
                                  col(bbn[2]), h1, h2, batch2d,
                                  s0, s1, s2, Wp, bp4)
    return node_poolT.T, gpool
